# baseline (speedup 1.0000x reference)
import functools

import jax
import jax.numpy as jnp
from jax import lax
from jax.experimental import pallas as pl
from jax.experimental.pallas import tpu as pltpu
from jax.experimental.pallas import tpu_sc as plsc

B = 4
S = 2048
H = 768
E = 64
TOPK = 2
NUM_CLASSES = 7
OUTC = NUM_CLASSES + 1

NW = 32
TOK = B * S
TPW = TOK // NW
WPB = S // TPW
CH = 64
NCH = TPW // CH


def _sc_gather(ids, table):
    info = plsc.get_sparse_core_info()
    ncores = info.num_cores
    mesh = plsc.VectorSubcoreMesh(core_axis_name="c", subcore_axis_name="s")

    @functools.partial(
        pl.kernel,
        mesh=mesh,
        out_type=jax.ShapeDtypeStruct((TOK, H), jnp.float32),
        scratch_types=[
            pltpu.VMEM((TPW,), jnp.int32),
            pltpu.VMEM((2, CH, H), jnp.float32),
            pltpu.SemaphoreType.DMA,
            pltpu.SemaphoreType.DMA,
        ],
    )
    def gather_kernel(ids_hbm, table_hbm, out_hbm, idx_v, rows_v, gsem, ssem):
        wid = lax.axis_index("s") * ncores + lax.axis_index("c")
        b = wid // WPB
        col0 = (wid % WPB) * TPW
        base = wid * TPW
        pltpu.sync_copy(ids_hbm.at[b, pl.ds(col0, TPW)], idx_v)
        g_prev = pltpu.async_copy(
            table_hbm.at[idx_v.at[pl.ds(0, CH)]], rows_v.at[0], gsem
        )
        s_prev = None
        for c in range(NCH):
            if c + 1 < NCH:
                g_next = pltpu.async_copy(
                    table_hbm.at[idx_v.at[pl.ds((c + 1) * CH, CH)]],
                    rows_v.at[(c + 1) % 2],
                    gsem,
                )
            g_prev.wait()
            if s_prev is not None:
                s_prev.wait()
            s_prev = pltpu.async_copy(
                rows_v.at[c % 2], out_hbm.at[pl.ds(base + c * CH, CH)], ssem
            )
            if c + 1 < NCH:
                g_prev = g_next
        s_prev.wait()

    return gather_kernel(ids, table)


def _tc_gate_moe(emb3, Wg, bg2, experts_W, experts_b3, C, d2):
    def gm_kernel(emb_ref, wg_ref, bg_ref, W_hbm, b_ref, C_ref, d_ref,
                  emo_ref, trig_ref, wscr, sem):
        eb = emb_ref[0]
        ones = jnp.full((1, S), 1.0 / S, dtype=jnp.float32)
        pooled = jnp.dot(ones, eb, preferred_element_type=jnp.float32)
        g = (
            jnp.dot(pooled, wg_ref[...], preferred_element_type=jnp.float32)
            + bg_ref[...]
        )
        m = jnp.max(g, axis=-1, keepdims=True)
        ex = jnp.exp(g - m)
        p = ex / jnp.sum(ex, axis=-1, keepdims=True)
        iota = lax.broadcasted_iota(jnp.int32, (1, E), 1)
        w1 = jnp.max(p)
        i1 = jnp.min(jnp.where(p == w1, iota, E))
        p2 = jnp.where(iota == i1, -jnp.inf, p)
        w2 = jnp.max(p2)
        i2 = jnp.min(jnp.where(p2 == w2, iota, E))
        cp1 = pltpu.make_async_copy(W_hbm.at[i1], wscr.at[0], sem.at[0])
        cp2 = pltpu.make_async_copy(W_hbm.at[i2], wscr.at[1], sem.at[1])
        cp1.start()
        cp2.start()
        qv = (
            jnp.dot(w1 * b_ref[i1] + w2 * b_ref[i2], C_ref[...],
                    preferred_element_type=jnp.float32)
            + d_ref[...]
        )
        cp1.wait()
        P = w1 * jnp.dot(wscr[0], C_ref[...], preferred_element_type=jnp.float32)
        cp2.wait()
        P = P + w2 * jnp.dot(wscr[1], C_ref[...], preferred_element_type=jnp.float32)
        v = jnp.dot(eb, P, preferred_element_type=jnp.float32) + qv
        emo_ref[0] = v[:, :NUM_CLASSES]
        trig_ref[0] = v[:, NUM_CLASSES:]

    return pl.pallas_call(
        gm_kernel,
        grid=(B,),
        in_specs=[
            pl.BlockSpec((1, S, H), lambda b: (b, 0, 0)),
            pl.BlockSpec((H, E), lambda b: (0, 0)),
            pl.BlockSpec((1, E), lambda b: (0, 0)),
            pl.BlockSpec(memory_space=pltpu.MemorySpace.HBM),
            pl.BlockSpec((E, 1, H), lambda b: (0, 0, 0)),
            pl.BlockSpec((H, OUTC), lambda b: (0, 0)),
            pl.BlockSpec((1, OUTC), lambda b: (0, 0)),
        ],
        out_specs=[
            pl.BlockSpec((1, S, NUM_CLASSES), lambda b: (b, 0, 0)),
            pl.BlockSpec((1, S, 1), lambda b: (b, 0, 0)),
        ],
        out_shape=[
            jax.ShapeDtypeStruct((B, S, NUM_CLASSES), jnp.float32),
            jax.ShapeDtypeStruct((B, S, 1), jnp.float32),
        ],
        scratch_shapes=[
            pltpu.VMEM((2, H, H), jnp.float32),
            pltpu.SemaphoreType.DMA((2,)),
        ],
    )(emb3, Wg, bg2, experts_W, experts_b3, C, d2)


def kernel(input_ids, attention_mask, emb_table, Wg, bg, experts_W, experts_b, We, be, Wt, bt):
    del attention_mask
    emb_flat = _sc_gather(input_ids.astype(jnp.int32), emb_table)
    emb3 = emb_flat.reshape(B, S, H)

    C = jnp.concatenate([We, Wt], axis=1)
    d2 = jnp.concatenate([be, bt]).reshape(1, OUTC)
    emotion_logits, trig3 = _tc_gate_moe(
        emb3, Wg, bg.reshape(1, E), experts_W, experts_b.reshape(E, 1, H), C, d2
    )
    return (emotion_logits, trig3.reshape(B, S))


# speedup vs baseline: 1.0273x; 1.0273x over previous
import functools

import jax
import jax.numpy as jnp
from jax import lax
from jax.experimental import pallas as pl
from jax.experimental.pallas import tpu as pltpu
from jax.experimental.pallas import tpu_sc as plsc

B = 4
S = 2048
H = 768
E = 64
TOPK = 2
NUM_CLASSES = 7
OUTC = NUM_CLASSES + 1

NW = 32
TOK = B * S
TPW = TOK // NW
WPB = S // TPW
CH = 64
NCH = TPW // CH


def _sc_gather(ids, table):
    info = plsc.get_sparse_core_info()
    ncores = info.num_cores
    mesh = plsc.VectorSubcoreMesh(core_axis_name="c", subcore_axis_name="s")

    @functools.partial(
        pl.kernel,
        mesh=mesh,
        out_type=jax.ShapeDtypeStruct((TOK, H), jnp.float32),
        scratch_types=[
            pltpu.VMEM((TPW,), jnp.int32),
            pltpu.VMEM((2, CH, H), jnp.float32),
            pltpu.SemaphoreType.DMA,
            pltpu.SemaphoreType.DMA,
        ],
    )
    def gather_kernel(ids_hbm, table_hbm, out_hbm, idx_v, rows_v, gsem, ssem):
        wid = lax.axis_index("s") * ncores + lax.axis_index("c")
        b = wid // WPB
        col0 = (wid % WPB) * TPW
        base = wid * TPW
        pltpu.sync_copy(ids_hbm.at[b, pl.ds(col0, TPW)], idx_v)
        g_prev = pltpu.async_copy(
            table_hbm.at[idx_v.at[pl.ds(0, CH)]], rows_v.at[0], gsem
        )
        s_prev = None
        for c in range(NCH):
            if c + 1 < NCH:
                g_next = pltpu.async_copy(
                    table_hbm.at[idx_v.at[pl.ds((c + 1) * CH, CH)]],
                    rows_v.at[(c + 1) % 2],
                    gsem,
                )
            g_prev.wait()
            if s_prev is not None:
                s_prev.wait()
            s_prev = pltpu.async_copy(
                rows_v.at[c % 2], out_hbm.at[pl.ds(base + c * CH, CH)], ssem
            )
            if c + 1 < NCH:
                g_prev = g_next
        s_prev.wait()

    return gather_kernel(ids, table)


def _tc_gate_moe(emb3, Wg, bg2, experts_W, experts_b3, We, be2, WtT, bt2):
    _dn_t = (((1,), (1,)), ((), ()))

    def gm_kernel(emb_ref, wg_ref, bg_ref, W_hbm, b_ref, we_ref, be_ref,
                  wtT_ref, bt_ref, emo_ref, trig_ref, wscr, sem):
        bidx = pl.program_id(0)
        eb = emb_ref[0]
        ones = jnp.full((1, S), 1.0 / S, dtype=jnp.float32)
        pooled = jnp.dot(ones, eb, preferred_element_type=jnp.float32)
        g = (
            jnp.dot(pooled, wg_ref[...], preferred_element_type=jnp.float32)
            + bg_ref[...]
        )
        m = jnp.max(g, axis=-1, keepdims=True)
        ex = jnp.exp(g - m)
        p = ex / jnp.sum(ex, axis=-1, keepdims=True)
        iota = lax.broadcasted_iota(jnp.int32, (1, E), 1)
        w1 = jnp.max(p)
        i1 = jnp.min(jnp.where(p == w1, iota, E))
        p2 = jnp.where(iota == i1, -jnp.inf, p)
        w2 = jnp.max(p2)
        i2 = jnp.min(jnp.where(p2 == w2, iota, E))
        cp1 = pltpu.make_async_copy(W_hbm.at[i1], wscr.at[0], sem.at[0])
        cp2 = pltpu.make_async_copy(W_hbm.at[i2], wscr.at[1], sem.at[1])
        cp1.start()
        cp2.start()
        bb = w1 * b_ref[i1] + w2 * b_ref[i2]
        qe = (
            jnp.dot(bb, we_ref[...], preferred_element_type=jnp.float32)
            + be_ref[...]
        )
        tb = lax.dot_general(bb, wtT_ref[...], _dn_t,
                             preferred_element_type=jnp.float32)
        tbs = tb[0, 0] + bt_ref[0, 0]
        cp1.wait()
        Pe = w1 * jnp.dot(wscr[0], we_ref[...], preferred_element_type=jnp.float32)
        rP = w1 * lax.dot_general(wtT_ref[...], wscr[0], _dn_t,
                                  preferred_element_type=jnp.float32)
        cp2.wait()
        Pe = Pe + w2 * jnp.dot(wscr[1], we_ref[...], preferred_element_type=jnp.float32)
        rP = rP + w2 * lax.dot_general(wtT_ref[...], wscr[1], _dn_t,
                                       preferred_element_type=jnp.float32)
        emo_ref[0] = (
            jnp.dot(eb, Pe, preferred_element_type=jnp.float32) + qe
        )
        trig_row = (
            lax.dot_general(rP, eb, _dn_t, preferred_element_type=jnp.float32)
            + tbs
        )
        trig_ref[pl.ds(bidx, 1), :] = trig_row

    return pl.pallas_call(
        gm_kernel,
        grid=(B,),
        in_specs=[
            pl.BlockSpec((1, S, H), lambda b: (b, 0, 0)),
            pl.BlockSpec((H, E), lambda b: (0, 0)),
            pl.BlockSpec((1, E), lambda b: (0, 0)),
            pl.BlockSpec(memory_space=pltpu.MemorySpace.HBM),
            pl.BlockSpec((E, 1, H), lambda b: (0, 0, 0)),
            pl.BlockSpec((H, NUM_CLASSES), lambda b: (0, 0)),
            pl.BlockSpec((1, NUM_CLASSES), lambda b: (0, 0)),
            pl.BlockSpec((1, H), lambda b: (0, 0)),
            pl.BlockSpec((1, 1), lambda b: (0, 0)),
        ],
        out_specs=[
            pl.BlockSpec((1, S, NUM_CLASSES), lambda b: (b, 0, 0)),
            pl.BlockSpec((B, S), lambda b: (0, 0)),
        ],
        out_shape=[
            jax.ShapeDtypeStruct((B, S, NUM_CLASSES), jnp.float32),
            jax.ShapeDtypeStruct((B, S), jnp.float32),
        ],
        scratch_shapes=[
            pltpu.VMEM((2, H, H), jnp.float32),
            pltpu.SemaphoreType.DMA((2,)),
        ],
    )(emb3, Wg, bg2, experts_W, experts_b3, We, be2, WtT, bt2)


def kernel(input_ids, attention_mask, emb_table, Wg, bg, experts_W, experts_b, We, be, Wt, bt):
    del attention_mask
    emb_flat = _sc_gather(input_ids.astype(jnp.int32), emb_table)
    emb3 = emb_flat.reshape(B, S, H)

    emotion_logits, trigger_logits = _tc_gate_moe(
        emb3,
        Wg,
        bg.reshape(1, E),
        experts_W,
        experts_b.reshape(E, 1, H),
        We,
        be.reshape(1, NUM_CLASSES),
        Wt.reshape(1, H),
        bt.reshape(1, 1),
    )
    return (emotion_logits, trigger_logits)


# speedup vs baseline: 1.0399x; 1.0123x over previous
import functools

import jax
import jax.numpy as jnp
from jax import lax
from jax.experimental import pallas as pl
from jax.experimental.pallas import tpu as pltpu
from jax.experimental.pallas import tpu_sc as plsc

B = 4
S = 2048
H = 768
E = 64
TOPK = 2
NUM_CLASSES = 7
OUTC = NUM_CLASSES + 1

NW = 32
TOK = B * S
TPW = TOK // NW
WPB = S // TPW
CH = 64
NCH = TPW // CH


def _sc_gather(ids, table):
    info = plsc.get_sparse_core_info()
    ncores = info.num_cores
    mesh = plsc.VectorSubcoreMesh(core_axis_name="c", subcore_axis_name="s")

    @functools.partial(
        pl.kernel,
        mesh=mesh,
        out_type=jax.ShapeDtypeStruct((TOK, H), jnp.float32),
        scratch_types=[
            pltpu.VMEM((TPW,), jnp.int32),
            pltpu.VMEM((2, CH, H), jnp.float32),
            pltpu.SemaphoreType.DMA,
            pltpu.SemaphoreType.DMA,
        ],
    )
    def gather_kernel(ids_hbm, table_hbm, out_hbm, idx_v, rows_v, gsem, ssem):
        wid = lax.axis_index("s") * ncores + lax.axis_index("c")
        b = wid // WPB
        col0 = (wid % WPB) * TPW
        base = wid * TPW
        pltpu.sync_copy(ids_hbm.at[b, pl.ds(col0, TPW)], idx_v)
        g_prev = pltpu.async_copy(
            table_hbm.at[idx_v.at[pl.ds(0, CH)]], rows_v.at[0], gsem
        )
        s_prev = None
        for c in range(NCH):
            if c + 1 < NCH:
                g_next = pltpu.async_copy(
                    table_hbm.at[idx_v.at[pl.ds((c + 1) * CH, CH)]],
                    rows_v.at[(c + 1) % 2],
                    gsem,
                )
            g_prev.wait()
            if s_prev is not None:
                s_prev.wait()
            s_prev = pltpu.async_copy(
                rows_v.at[c % 2], out_hbm.at[pl.ds(base + c * CH, CH)], ssem
            )
            if c + 1 < NCH:
                g_prev = g_next
        s_prev.wait()

    return gather_kernel(ids, table)


def _tc_gate_moe(emb3, Wg, bg2, experts_W, experts_b3, We, be2, WtT, bt2):
    _dn_t = (((1,), (1,)), ((), ()))

    def gm_kernel(emb_ref, wg_ref, bg_ref, W_hbm, b_ref, we_ref, be_ref,
                  wtT_ref, bt_ref, emo_ref, trig_ref, wscr, sem):
        bidx = pl.program_id(0)
        eb = emb_ref[0]
        pooled = jnp.sum(eb, axis=0, keepdims=True) * (1.0 / S)
        g = (
            jnp.dot(pooled, wg_ref[...], preferred_element_type=jnp.float32)
            + bg_ref[...]
        )
        m = jnp.max(g, axis=-1, keepdims=True)
        ex = jnp.exp(g - m)
        p = ex / jnp.sum(ex, axis=-1, keepdims=True)
        iota = lax.broadcasted_iota(jnp.int32, (1, E), 1)
        w1 = jnp.max(p)
        i1 = jnp.min(jnp.where(p == w1, iota, E))
        p2 = jnp.where(iota == i1, -jnp.inf, p)
        w2 = jnp.max(p2)
        i2 = jnp.min(jnp.where(p2 == w2, iota, E))
        cp1 = pltpu.make_async_copy(W_hbm.at[i1], wscr.at[0], sem.at[0])
        cp2 = pltpu.make_async_copy(W_hbm.at[i2], wscr.at[1], sem.at[1])
        cp1.start()
        cp2.start()
        bb = w1 * b_ref[i1] + w2 * b_ref[i2]
        qe = (
            jnp.dot(bb, we_ref[...], preferred_element_type=jnp.float32)
            + be_ref[...]
        )
        tb = lax.dot_general(bb, wtT_ref[...], _dn_t,
                             preferred_element_type=jnp.float32)
        tbs = tb[0, 0] + bt_ref[0, 0]
        cp1.wait()
        Pe = w1 * jnp.dot(wscr[0], we_ref[...], preferred_element_type=jnp.float32)
        rP = w1 * lax.dot_general(wtT_ref[...], wscr[0], _dn_t,
                                  preferred_element_type=jnp.float32)
        cp2.wait()
        Pe = Pe + w2 * jnp.dot(wscr[1], we_ref[...], preferred_element_type=jnp.float32)
        rP = rP + w2 * lax.dot_general(wtT_ref[...], wscr[1], _dn_t,
                                       preferred_element_type=jnp.float32)
        emo_ref[0] = (
            jnp.dot(eb, Pe, preferred_element_type=jnp.float32) + qe
        )
        trig_row = (
            lax.dot_general(rP, eb, _dn_t, preferred_element_type=jnp.float32)
            + tbs
        )
        trig_ref[pl.ds(bidx, 1), :] = trig_row

    return pl.pallas_call(
        gm_kernel,
        grid=(B,),
        in_specs=[
            pl.BlockSpec((1, S, H), lambda b: (b, 0, 0)),
            pl.BlockSpec((H, E), lambda b: (0, 0)),
            pl.BlockSpec((1, E), lambda b: (0, 0)),
            pl.BlockSpec(memory_space=pltpu.MemorySpace.HBM),
            pl.BlockSpec((E, 1, H), lambda b: (0, 0, 0)),
            pl.BlockSpec((H, NUM_CLASSES), lambda b: (0, 0)),
            pl.BlockSpec((1, NUM_CLASSES), lambda b: (0, 0)),
            pl.BlockSpec((1, H), lambda b: (0, 0)),
            pl.BlockSpec((1, 1), lambda b: (0, 0)),
        ],
        out_specs=[
            pl.BlockSpec((1, S, NUM_CLASSES), lambda b: (b, 0, 0)),
            pl.BlockSpec((B, S), lambda b: (0, 0)),
        ],
        out_shape=[
            jax.ShapeDtypeStruct((B, S, NUM_CLASSES), jnp.float32),
            jax.ShapeDtypeStruct((B, S), jnp.float32),
        ],
        scratch_shapes=[
            pltpu.VMEM((2, H, H), jnp.float32),
            pltpu.SemaphoreType.DMA((2,)),
        ],
    )(emb3, Wg, bg2, experts_W, experts_b3, We, be2, WtT, bt2)


def kernel(input_ids, attention_mask, emb_table, Wg, bg, experts_W, experts_b, We, be, Wt, bt):
    del attention_mask
    emb_flat = _sc_gather(input_ids.astype(jnp.int32), emb_table)
    emb3 = emb_flat.reshape(B, S, H)

    emotion_logits, trigger_logits = _tc_gate_moe(
        emb3,
        Wg,
        bg.reshape(1, E),
        experts_W,
        experts_b.reshape(E, 1, H),
        We,
        be.reshape(1, NUM_CLASSES),
        Wt.reshape(1, H),
        bt.reshape(1, 1),
    )
    return (emotion_logits, trigger_logits)


# speedup vs baseline: 1.1315x; 1.0880x over previous
import functools

import jax
import jax.numpy as jnp
from jax import lax
from jax.experimental import pallas as pl
from jax.experimental.pallas import tpu as pltpu
from jax.experimental.pallas import tpu_sc as plsc

B = 4
S = 2048
H = 768
E = 64
TOPK = 2
NUM_CLASSES = 7

NW = 32
TOK = B * S
TPW = TOK // NW
WPB = S // TPW
CH = 64
NCH = TPW // CH
LANES = 16
NV = H // LANES
NPASS = 3
NVP = NV // NPASS


def _sc_gather(ids, table):
    info = plsc.get_sparse_core_info()
    ncores = info.num_cores
    mesh = plsc.VectorSubcoreMesh(core_axis_name="c", subcore_axis_name="s")

    @functools.partial(
        pl.kernel,
        mesh=mesh,
        out_type=[
            jax.ShapeDtypeStruct((TOK, H), jnp.float32),
            jax.ShapeDtypeStruct((NW, H), jnp.float32),
        ],
        scratch_types=[
            pltpu.VMEM((TPW,), jnp.int32),
            pltpu.VMEM((2, CH, H), jnp.float32),
            pltpu.VMEM((H,), jnp.float32),
            pltpu.SemaphoreType.DMA,
            pltpu.SemaphoreType.DMA,
        ],
    )
    def gather_kernel(ids_hbm, table_hbm, out_hbm, pool_hbm,
                      idx_v, rows_v, acc_v, gsem, ssem):
        wid = lax.axis_index("s") * ncores + lax.axis_index("c")
        b = wid // WPB
        col0 = (wid % WPB) * TPW
        base = wid * TPW
        pltpu.sync_copy(ids_hbm.at[b, pl.ds(col0, TPW)], idx_v)
        acc = [jnp.zeros((LANES,), jnp.float32) for _ in range(NV)]
        g_prev = pltpu.async_copy(
            table_hbm.at[idx_v.at[pl.ds(0, CH)]], rows_v.at[0], gsem
        )
        s_prev = None
        for c in range(NCH):
            buf = c % 2
            if c + 1 < NCH:
                g_next = pltpu.async_copy(
                    table_hbm.at[idx_v.at[pl.ds((c + 1) * CH, CH)]],
                    rows_v.at[(c + 1) % 2],
                    gsem,
                )
            g_prev.wait()
            if s_prev is not None:
                s_prev.wait()
            s_prev = pltpu.async_copy(
                rows_v.at[buf], out_hbm.at[pl.ds(base + c * CH, CH)], ssem
            )
            for kk in range(NPASS):
                def body(r, carry, _kk=kk, _buf=buf):
                    return tuple(
                        carry[j]
                        + rows_v[_buf, r, pl.ds((_kk * NVP + j) * LANES, LANES)]
                        for j in range(NVP)
                    )
                sub = lax.fori_loop(
                    0, CH, body, tuple(acc[kk * NVP:(kk + 1) * NVP])
                )
                acc[kk * NVP:(kk + 1) * NVP] = list(sub)
        for j in range(NV):
            acc_v[pl.ds(j * LANES, LANES)] = acc[j]
        s_prev.wait()
        pltpu.sync_copy(acc_v, pool_hbm.at[wid])

    return gather_kernel(ids, table)


def _tc_gate(pool_part, Wg, bg2):
    def gate_kernel(pp_ref, wg_ref, bg_ref, wout_ref, iout_ref):
        pr = pp_ref[...]
        pooled = jnp.concatenate(
            [
                jnp.sum(pr[b * WPB:(b + 1) * WPB], axis=0, keepdims=True)
                for b in range(B)
            ],
            axis=0,
        ) * (1.0 / S)
        g = (
            jnp.dot(pooled, wg_ref[...], preferred_element_type=jnp.float32)
            + bg_ref[...]
        )
        m = jnp.max(g, axis=-1, keepdims=True)
        ex = jnp.exp(g - m)
        p = ex / jnp.sum(ex, axis=-1, keepdims=True)
        iota = lax.broadcasted_iota(jnp.int32, (B, E), 1)
        w1 = jnp.max(p, axis=-1, keepdims=True)
        i1 = jnp.min(jnp.where(p == w1, iota, E), axis=-1, keepdims=True)
        p2 = jnp.where(iota == i1, -jnp.inf, p)
        w2 = jnp.max(p2, axis=-1, keepdims=True)
        i2 = jnp.min(jnp.where(p2 == w2, iota, E), axis=-1, keepdims=True)
        lane = lax.broadcasted_iota(jnp.int32, (B, 128), 1)
        wout_ref[...] = jnp.where(lane == 0, w1, jnp.where(lane == 1, w2, 0.0))
        iout_ref[...] = jnp.where(lane == 0, i1, jnp.where(lane == 1, i2, 0))

    return pl.pallas_call(
        gate_kernel,
        out_shape=[
            jax.ShapeDtypeStruct((B, 128), jnp.float32),
            jax.ShapeDtypeStruct((B, 128), jnp.int32),
        ],
    )(pool_part, Wg, bg2)


def _tc_moe(topk_i, topk_w, emb3, experts_W, experts_b3, We, be2, WtT, bt2):
    _dn_t = (((1,), (1,)), ((), ()))

    def moe_kernel(i_ref, w_ref, emb_ref, W1_ref, W2_ref, b1_ref, b2_ref,
                   we_ref, be_ref, wtT_ref, bt_ref, emo_ref, trig_ref):
        bidx = pl.program_id(0)
        w1 = w_ref[bidx, 0]
        w2 = w_ref[bidx, 1]
        eb = emb_ref[0]
        bb = w1 * b1_ref[0] + w2 * b2_ref[0]
        qe = (
            jnp.dot(bb, we_ref[...], preferred_element_type=jnp.float32)
            + be_ref[...]
        )
        tb = lax.dot_general(bb, wtT_ref[...], _dn_t,
                             preferred_element_type=jnp.float32)
        tbs = tb[0, 0] + bt_ref[0, 0]
        Pe = w1 * jnp.dot(W1_ref[0], we_ref[...], preferred_element_type=jnp.float32)
        Pe = Pe + w2 * jnp.dot(W2_ref[0], we_ref[...], preferred_element_type=jnp.float32)
        rP = w1 * lax.dot_general(wtT_ref[...], W1_ref[0], _dn_t,
                                  preferred_element_type=jnp.float32)
        rP = rP + w2 * lax.dot_general(wtT_ref[...], W2_ref[0], _dn_t,
                                       preferred_element_type=jnp.float32)
        emo_ref[0] = (
            jnp.dot(eb, Pe, preferred_element_type=jnp.float32) + qe
        )
        trig_row = (
            lax.dot_general(rP, eb, _dn_t, preferred_element_type=jnp.float32)
            + tbs
        )
        trig_ref[pl.ds(bidx, 1), :] = trig_row

    grid_spec = pltpu.PrefetchScalarGridSpec(
        num_scalar_prefetch=2,
        grid=(B,),
        in_specs=[
            pl.BlockSpec((1, S, H), lambda b, i, w: (b, 0, 0)),
            pl.BlockSpec((1, H, H), lambda b, i, w: (i[b, 0], 0, 0)),
            pl.BlockSpec((1, H, H), lambda b, i, w: (i[b, 1], 0, 0)),
            pl.BlockSpec((1, 1, H), lambda b, i, w: (i[b, 0], 0, 0)),
            pl.BlockSpec((1, 1, H), lambda b, i, w: (i[b, 1], 0, 0)),
            pl.BlockSpec((H, NUM_CLASSES), lambda b, i, w: (0, 0)),
            pl.BlockSpec((1, NUM_CLASSES), lambda b, i, w: (0, 0)),
            pl.BlockSpec((1, H), lambda b, i, w: (0, 0)),
            pl.BlockSpec((1, 1), lambda b, i, w: (0, 0)),
        ],
        out_specs=[
            pl.BlockSpec((1, S, NUM_CLASSES), lambda b, i, w: (b, 0, 0)),
            pl.BlockSpec((B, S), lambda b, i, w: (0, 0)),
        ],
    )
    return pl.pallas_call(
        moe_kernel,
        grid_spec=grid_spec,
        out_shape=[
            jax.ShapeDtypeStruct((B, S, NUM_CLASSES), jnp.float32),
            jax.ShapeDtypeStruct((B, S), jnp.float32),
        ],
        compiler_params=pltpu.CompilerParams(dimension_semantics=("arbitrary",)),
    )(topk_i, topk_w, emb3, experts_W, experts_W, experts_b3, experts_b3,
      We, be2, WtT, bt2)


def kernel(input_ids, attention_mask, emb_table, Wg, bg, experts_W, experts_b, We, be, Wt, bt):
    del attention_mask
    emb_flat, pool_part = _sc_gather(input_ids.astype(jnp.int32), emb_table)
    emb3 = emb_flat.reshape(B, S, H)

    topk_w, topk_i = _tc_gate(pool_part, Wg, bg.reshape(1, E))
    emotion_logits, trigger_logits = _tc_moe(
        topk_i,
        topk_w,
        emb3,
        experts_W,
        experts_b.reshape(E, 1, H),
        We,
        be.reshape(1, NUM_CLASSES),
        Wt.reshape(1, H),
        bt.reshape(1, 1),
    )
    return (emotion_logits, trigger_logits)
